# Initial kernel scaffold; baseline (speedup 1.0000x reference)
import jax
import jax.numpy as jnp
from jax.experimental import pallas as pl


def kernel(mol_a_reprs, node_feats, edge_feats, edge_index, node_graph_ids, W_node, b_node, W_edge, b_edge, W_msg, b_msg, W_upd, b_upd, W_ih, W_hh, b_ih, b_hh, W1, b1, W2, b2):
    raise NotImplementedError("write your pallas kernel here")


# speedup vs baseline: 2.0141x; 2.0141x over previous
import functools
import jax
import jax.numpy as jnp
from jax import lax
from jax.experimental import pallas as pl
from jax.experimental.pallas import tpu as pltpu
from jax.experimental.pallas import tpu_sc as plsc

N_MPN_STEPS = 4
N_RNN_ITERS = 2

B, MA = 1024, 128
N, E = 25600, 51200
NF, EF = 64, 16
D, EH = 64, 32
H, MH = 64, 128
NPAD = 8 * N + B


def _node_embed_body(nf_ref, wn_ref, bn_ref, wmt_ref, nh_ref, p_ref):
    nh = jnp.maximum(nf_ref[...] @ wn_ref[...] + bn_ref[...], 0.0)
    nh_ref[...] = nh
    p_ref[...] = nh @ wmt_ref[...]


def _node_embed(nf, wn, bn, wmt, blk=1024):
    g = N // blk
    return pl.pallas_call(
        _node_embed_body,
        grid=(g,),
        in_specs=[
            pl.BlockSpec((blk, NF), lambda i: (i, 0)),
            pl.BlockSpec((NF, D), lambda i: (0, 0)),
            pl.BlockSpec((1, D), lambda i: (0, 0)),
            pl.BlockSpec((D, D), lambda i: (0, 0)),
        ],
        out_specs=[
            pl.BlockSpec((blk, D), lambda i: (i, 0)),
            pl.BlockSpec((blk, D), lambda i: (i, 0)),
        ],
        out_shape=[
            jax.ShapeDtypeStruct((N, D), jnp.float32),
            jax.ShapeDtypeStruct((N, D), jnp.float32),
        ],
    )(nf, wn, bn.reshape(1, D), wmt)


def _edge_embed_body(ef_ref, we_ref, be_ref, wmb_ref, bm_ref, ehp_ref):
    eh = jnp.maximum(ef_ref[...] @ we_ref[...] + be_ref[...], 0.0)
    ehp_ref[...] = eh @ wmb_ref[...] + bm_ref[...]


def _edge_embed(ef, we, be, wmb, bm, blk=2048):
    g = E // blk
    return pl.pallas_call(
        _edge_embed_body,
        grid=(g,),
        in_specs=[
            pl.BlockSpec((blk, EF), lambda i: (i, 0)),
            pl.BlockSpec((EF, EH), lambda i: (0, 0)),
            pl.BlockSpec((1, EH), lambda i: (0, 0)),
            pl.BlockSpec((EH, D), lambda i: (0, 0)),
            pl.BlockSpec((1, D), lambda i: (0, 0)),
        ],
        out_specs=pl.BlockSpec((blk, D), lambda i: (i, 0)),
        out_shape=jax.ShapeDtypeStruct((E, D), jnp.float32),
    )(ef, we, be.reshape(1, EH), wmb, bm.reshape(1, D))


def _node_update_body(nh_ref, agg_ref, wu1_ref, wu2_ref, bu_ref, w2_ref,
                      b2_ref, nh_out_ref, sec_ref, *, with_bias):
    agg = agg_ref[0] + agg_ref[1]
    nh = jnp.maximum(nh_ref[...] @ wu1_ref[...] + agg @ wu2_ref[...]
                     + bu_ref[...], 0.0)
    nh_out_ref[...] = nh
    sec = nh @ w2_ref[...]
    if with_bias:
        sec = sec + b2_ref[...]
    sec_ref[...] = sec


def _node_update(nh, aggp, wu1, wu2, bu, w2, b2, *, with_bias, blk=1024):
    g = N // blk
    k = w2.shape[1]
    return pl.pallas_call(
        functools.partial(_node_update_body, with_bias=with_bias),
        grid=(g,),
        in_specs=[
            pl.BlockSpec((blk, D), lambda i: (i, 0)),
            pl.BlockSpec((2, blk, D), lambda i: (0, i, 0)),
            pl.BlockSpec((D, D), lambda i: (0, 0)),
            pl.BlockSpec((D, D), lambda i: (0, 0)),
            pl.BlockSpec((1, D), lambda i: (0, 0)),
            pl.BlockSpec((D, k), lambda i: (0, 0)),
            pl.BlockSpec((1, k), lambda i: (0, 0)),
        ],
        out_specs=[
            pl.BlockSpec((blk, D), lambda i: (i, 0)),
            pl.BlockSpec((blk, k), lambda i: (i, 0)),
        ],
        out_shape=[
            jax.ShapeDtypeStruct((N, D), jnp.float32),
            jax.ShapeDtypeStruct((N, k), jnp.float32),
        ],
    )(nh, aggp, wu1, wu2, bu.reshape(1, D), w2, b2.reshape(1, k))


def _mol_proj_body(mol_ref, w_ref, b_ref, out_ref):
    out_ref[...] = mol_ref[...] @ w_ref[...] + b_ref[...]


def _mol_proj(mol, w1a, b1):
    return pl.pallas_call(
        _mol_proj_body,
        in_specs=[
            pl.BlockSpec((B, MA), lambda: (0, 0)),
            pl.BlockSpec((MA, MH), lambda: (0, 0)),
            pl.BlockSpec((1, MH), lambda: (0, 0)),
        ],
        out_specs=pl.BlockSpec((B, MH), lambda: (0, 0)),
        out_shape=jax.ShapeDtypeStruct((B, MH), jnp.float32),
    )(mol, w1a, b1.reshape(1, MH))


NC, NS, LANES = 2, 16, 16
NW = NC * NS
_SC_MESH = dict(core_axis_name="c", subcore_axis_name="s")


def _worker_id():
    return lax.axis_index("c") * NS + lax.axis_index("s")


_E_PER_W = E // NW
_ECHUNK = 64
_N_ECHUNKS = _E_PER_W // _ECHUNK


def _mpn_edge_body(p_hbm, ehp_hbm, src_hbm, dst_hbm, aggp_hbm,
                   idxs_v, idxd_v, rows_v, ehp_v, zrow_v, agg_sh, sem):
    c = lax.axis_index("c")
    s = lax.axis_index("s")
    w = c * NS + s
    nrows = N // NS

    def zero_body(j, _):
        zrow_v[j // (D // LANES), pl.ds((j % (D // LANES)) * LANES, LANES)] = (
            jnp.zeros((LANES,), jnp.float32))
        return 0
    lax.fori_loop(0, _ECHUNK * D // LANES, zero_body, 0)
    for k in range(nrows // _ECHUNK):
        pltpu.sync_copy(zrow_v, agg_sh.at[pl.ds(s * nrows + k * _ECHUNK,
                                                _ECHUNK)])
    plsc.subcore_barrier()

    def chunk_body(k, _):
        base = w * _E_PER_W + k * _ECHUNK
        pltpu.sync_copy(src_hbm.at[pl.ds(base, _ECHUNK)], idxs_v)
        pltpu.async_copy(p_hbm.at[idxs_v], rows_v, sem).wait()
        pltpu.sync_copy(ehp_hbm.at[pl.ds(base, _ECHUNK)], ehp_v)

        def ew(j, _):
            r = j // (D // LANES)
            l = (j % (D // LANES)) * LANES
            rows_v[r, pl.ds(l, LANES)] = jnp.maximum(
                rows_v[r, pl.ds(l, LANES)] + ehp_v[r, pl.ds(l, LANES)], 0.0)
            return 0
        lax.fori_loop(0, _ECHUNK * D // LANES, ew, 0)

        pltpu.sync_copy(dst_hbm.at[pl.ds(base, _ECHUNK)], idxd_v)
        pltpu.sync_copy(rows_v, agg_sh.at[idxd_v], add=True)
        return 0
    lax.fori_loop(0, _N_ECHUNKS, chunk_body, 0)
    plsc.subcore_barrier()

    for k in range(nrows // _ECHUNK):
        r0 = s * nrows + k * _ECHUNK
        pltpu.sync_copy(agg_sh.at[pl.ds(r0, _ECHUNK)],
                        aggp_hbm.at[c, pl.ds(r0, _ECHUNK)])


def _mpn_edge_pass(p, ehp, src, dst):
    f = pl.kernel(
        _mpn_edge_body,
        out_type=jax.ShapeDtypeStruct((NC, N, D), jnp.float32),
        mesh=plsc.VectorSubcoreMesh(**_SC_MESH),
        compiler_params=pltpu.CompilerParams(use_tc_tiling_on_sc=False, needs_layout_passes=False),
        scratch_types=[
            pltpu.VMEM((_ECHUNK,), jnp.int32),
            pltpu.VMEM((_ECHUNK,), jnp.int32),
            pltpu.VMEM((_ECHUNK, D), jnp.float32),
            pltpu.VMEM((_ECHUNK, D), jnp.float32),
            pltpu.VMEM((_ECHUNK, D), jnp.float32),
            pltpu.VMEM_SHARED((N, D), jnp.float32),
            pltpu.SemaphoreType.DMA,
        ],
    )
    return f(p, ehp, src, dst)


_N_PER_W = N // NW
_RCHUNK = 80
_N_RCHUNKS = _N_PER_W // _RCHUNK
_B_PER_W = B // NW


def _reorder_body(gi_hbm, pos_hbm, xc_hbm, idx_v, rows_v, sem):
    w = _worker_id()

    def chunk_body(k, _):
        base = w * _N_PER_W + k * _RCHUNK
        pltpu.sync_copy(gi_hbm.at[pl.ds(base, _RCHUNK)], rows_v)
        pltpu.sync_copy(pos_hbm.at[pl.ds(base, _RCHUNK)], idx_v)
        pltpu.async_copy(rows_v, xc_hbm.at[idx_v], sem).wait()
        return 0
    lax.fori_loop(0, _N_RCHUNKS, chunk_body, 0)


def _reorder(gi, pos):
    f = pl.kernel(
        _reorder_body,
        out_type=jax.ShapeDtypeStruct((NPAD, 3 * H), jnp.float32),
        mesh=plsc.VectorSubcoreMesh(**_SC_MESH),
        compiler_params=pltpu.CompilerParams(use_tc_tiling_on_sc=False, needs_layout_passes=False),
        scratch_types=[
            pltpu.VMEM((_RCHUNK,), jnp.int32),
            pltpu.VMEM((_RCHUNK, 3 * H), jnp.float32),
            pltpu.SemaphoreType.DMA,
        ],
    )
    return f(gi, pos)


def _final_gather_body(hc_hbm, pos_hbm, molp_hbm, gid_hbm, hn_hbm, mn_hbm,
                       idx_v, hbuf_v, mbuf_v, sem):
    w = _worker_id()

    def chunk_body(k, _):
        base = w * _N_PER_W + k * _RCHUNK
        pltpu.sync_copy(pos_hbm.at[pl.ds(base, _RCHUNK)], idx_v)
        pltpu.async_copy(hc_hbm.at[idx_v], hbuf_v, sem).wait()
        pltpu.sync_copy(hbuf_v, hn_hbm.at[pl.ds(base, _RCHUNK)])
        pltpu.sync_copy(gid_hbm.at[pl.ds(base, _RCHUNK)], idx_v)
        pltpu.async_copy(molp_hbm.at[idx_v], mbuf_v, sem).wait()
        pltpu.sync_copy(mbuf_v, mn_hbm.at[pl.ds(base, _RCHUNK)])
        return 0
    lax.fori_loop(0, _N_RCHUNKS, chunk_body, 0)


def _final_gather(hc, pos, molp, gid):
    f = pl.kernel(
        _final_gather_body,
        out_type=[
            jax.ShapeDtypeStruct((N, H), jnp.float32),
            jax.ShapeDtypeStruct((N, MH), jnp.float32),
        ],
        mesh=plsc.VectorSubcoreMesh(**_SC_MESH),
        compiler_params=pltpu.CompilerParams(use_tc_tiling_on_sc=False, needs_layout_passes=False),
        scratch_types=[
            pltpu.VMEM((_RCHUNK,), jnp.int32),
            pltpu.VMEM((_RCHUNK, H), jnp.float32),
            pltpu.VMEM((_RCHUNK, MH), jnp.float32),
            pltpu.SemaphoreType.DMA,
        ],
    )
    return f(hc, pos, molp, gid)


def _final_mlp_body(hn_ref, mn_ref, w1h_ref, w2_ref, b2_ref, out_ref):
    hid = jnp.maximum(mn_ref[...] + hn_ref[...] @ w1h_ref[...], 0.0)
    out_ref[...] = jax.nn.sigmoid(hid @ w2_ref[...] + b2_ref[...])


def _final_mlp(hn, mn, w1h, w2, b2, blk=1600):
    g = N // blk
    return pl.pallas_call(
        _final_mlp_body,
        grid=(g,),
        in_specs=[
            pl.BlockSpec((blk, H), lambda i: (i, 0)),
            pl.BlockSpec((blk, MH), lambda i: (i, 0)),
            pl.BlockSpec((H, MH), lambda i: (0, 0)),
            pl.BlockSpec((MH, 1), lambda i: (0, 0)),
            pl.BlockSpec((1, 1), lambda i: (0, 0)),
        ],
        out_specs=pl.BlockSpec((blk, 1), lambda i: (i, 0)),
        out_shape=jax.ShapeDtypeStruct((N, 1), jnp.float32),
    )(hn, mn, w1h, w2, b2.reshape(1, 1))


def _gru_body(maxi_ref, xc_ref, cnt_ref, whh_ref, bhh_ref, hc_ref,
              xbuf, h_ref, sem_i, sem_o):
    maxi = maxi_ref[0]
    h_ref[...] = jnp.zeros((B, H), jnp.float32)
    cnt = cnt_ref[...]
    whh = whh_ref[...]
    bhh = bhh_ref[...]

    def step(i, off, last):
        off = pl.multiple_of(off, 8)
        cp_in = pltpu.make_async_copy(xc_ref.at[pl.ds(off, B)], xbuf, sem_i)
        cp_in.start()
        h = h_ref[...]
        gh = h @ whh + bhh
        cp_in.wait()
        gi = xbuf[...]
        r = jax.nn.sigmoid(gi[:, :H] + gh[:, :H])
        z = jax.nn.sigmoid(gi[:, H:2 * H] + gh[:, H:2 * H])
        n = jnp.tanh(gi[:, 2 * H:] + r * gh[:, 2 * H:])
        newh = (1.0 - z) * n + z * h
        mask = i < cnt
        hnew = jnp.where(mask, newh, h)
        h_ref[...] = hnew
        if last:
            cp_out = pltpu.make_async_copy(h_ref, hc_ref.at[pl.ds(off, B)],
                                           sem_o)
            cp_out.start()
            cp_out.wait()
        na = jnp.sum(mask.astype(jnp.int32))
        return off + ((na + 7) // 8) * 8

    for t in range(N_RNN_ITERS + 1):
        lax.fori_loop(0, maxi,
                      functools.partial(step, last=(t == N_RNN_ITERS)), 0)


def _gru_phase(maxi, xc, counts_sorted, whh, bhh):
    return pl.pallas_call(
        _gru_body,
        in_specs=[
            pl.BlockSpec(memory_space=pltpu.SMEM),
            pl.BlockSpec(memory_space=pl.ANY),
            pl.BlockSpec((B, 1), lambda: (0, 0)),
            pl.BlockSpec((H, 3 * H), lambda: (0, 0)),
            pl.BlockSpec((1, 3 * H), lambda: (0, 0)),
        ],
        out_specs=pl.BlockSpec(memory_space=pl.ANY),
        out_shape=jax.ShapeDtypeStruct((NPAD, H), jnp.float32),
        scratch_shapes=[
            pltpu.VMEM((B, 3 * H), jnp.float32),
            pltpu.VMEM((B, H), jnp.float32),
            pltpu.SemaphoreType.DMA,
            pltpu.SemaphoreType.DMA,
        ],
    )(maxi.reshape(1), xc, counts_sorted.reshape(B, 1), whh,
      bhh.reshape(1, 3 * H))


def kernel(mol_a_reprs, node_feats, edge_feats, edge_index, node_graph_ids,
           W_node, b_node, W_edge, b_edge, W_msg, b_msg, W_upd, b_upd,
           W_ih, W_hh, b_ih, b_hh, W1, b1, W2, b2):
    src = edge_index[0]
    dst = edge_index[1]

    counts = jnp.bincount(node_graph_ids, length=B)
    offsets = jnp.cumsum(counts) - counts
    maxi = jnp.max(counts).astype(jnp.int32)
    pi = jnp.argsort(-counts)
    counts_sorted = counts[pi].astype(jnp.int32)
    invpi = jnp.zeros((B,), jnp.int32).at[pi].set(jnp.arange(B, dtype=jnp.int32))
    hist = jnp.bincount(counts, length=N + 1)
    a_per_step = (B - jnp.cumsum(hist)).astype(jnp.int32)
    a_al = ((a_per_step + 7) // 8) * 8
    step_off = jnp.cumsum(a_al) - a_al
    i_of_node = jnp.arange(N, dtype=jnp.int32) - offsets[node_graph_ids].astype(jnp.int32)
    pos = (step_off[i_of_node] + invpi[node_graph_ids]).astype(jnp.int32)

    wmt = W_msg[:D]
    wmb = W_msg[D:]
    wu1 = W_upd[:D]
    wu2 = W_upd[D:]
    w1a = W1[:MA]
    w1h = W1[MA:]

    nh, p = _node_embed(node_feats, W_node, b_node, wmt)
    ehp = _edge_embed(edge_feats, W_edge, b_edge, wmb, b_msg)
    for s in range(N_MPN_STEPS):
        aggp = _mpn_edge_pass(p, ehp, src, dst)
        last = s == N_MPN_STEPS - 1
        if last:
            nh, gi = _node_update(nh, aggp, wu1, wu2, b_upd, W_ih, b_ih,
                                  with_bias=True)
        else:
            nh, p = _node_update(nh, aggp, wu1, wu2, b_upd, wmt,
                                 jnp.zeros((D,), jnp.float32),
                                 with_bias=False)

    molp = _mol_proj(mol_a_reprs, w1a, b1)
    xc = _reorder(gi, pos)
    hc = _gru_phase(maxi, xc, counts_sorted, W_hh, b_hh)
    hn, mn = _final_gather(hc, pos, molp, node_graph_ids)
    return _final_mlp(hn, mn, w1h, W2, b2).reshape(N)


# speedup vs baseline: 2.3251x; 1.1545x over previous
import functools
import jax
import jax.numpy as jnp
from jax import lax
from jax.experimental import pallas as pl
from jax.experimental.pallas import tpu as pltpu
from jax.experimental.pallas import tpu_sc as plsc

N_MPN_STEPS = 4
N_RNN_ITERS = 2

B, MA = 1024, 128
N, E = 25600, 51200
NF, EF = 64, 16
D, EH = 64, 32
H, MH = 64, 128
NPAD = 8 * N + B
NT = N + 8


def _node_embed_body(nf_ref, wn_ref, bn_ref, wmt_ref, nh_ref, p_ref):
    nh = jnp.maximum(nf_ref[...] @ wn_ref[...] + bn_ref[...], 0.0)
    nh_ref[...] = nh
    p_ref[...] = nh @ wmt_ref[...]


def _node_embed(nf, wn, bn, wmt, blk=1024):
    g = N // blk
    return pl.pallas_call(
        _node_embed_body,
        grid=(g,),
        in_specs=[
            pl.BlockSpec((blk, NF), lambda i: (i, 0)),
            pl.BlockSpec((NF, D), lambda i: (0, 0)),
            pl.BlockSpec((1, D), lambda i: (0, 0)),
            pl.BlockSpec((D, D), lambda i: (0, 0)),
        ],
        out_specs=[
            pl.BlockSpec((blk, D), lambda i: (i, 0)),
            pl.BlockSpec((blk, D), lambda i: (i, 0)),
        ],
        out_shape=[
            jax.ShapeDtypeStruct((N, D), jnp.float32),
            jax.ShapeDtypeStruct((N, D), jnp.float32),
        ],
    )(nf, wn, bn.reshape(1, D), wmt)


def _edge_embed_body(ef_ref, we_ref, be_ref, wmb_ref, bm_ref, ehp_ref):
    eh = jnp.maximum(ef_ref[...] @ we_ref[...] + be_ref[...], 0.0)
    ehp_ref[...] = eh @ wmb_ref[...] + bm_ref[...]


def _edge_embed(ef, we, be, wmb, bm, blk=2048):
    g = E // blk
    return pl.pallas_call(
        _edge_embed_body,
        grid=(g,),
        in_specs=[
            pl.BlockSpec((blk, EF), lambda i: (i, 0)),
            pl.BlockSpec((EF, EH), lambda i: (0, 0)),
            pl.BlockSpec((1, EH), lambda i: (0, 0)),
            pl.BlockSpec((EH, D), lambda i: (0, 0)),
            pl.BlockSpec((1, D), lambda i: (0, 0)),
        ],
        out_specs=pl.BlockSpec((blk, D), lambda i: (i, 0)),
        out_shape=jax.ShapeDtypeStruct((E, D), jnp.float32),
    )(ef, we, be.reshape(1, EH), wmb, bm.reshape(1, D))


def _node_update_body(nh_ref, agg_ref, wu1_ref, wu2_ref, bu_ref, w2_ref,
                      b2_ref, nh_out_ref, sec_ref, *, with_bias):
    agg = agg_ref[0] + agg_ref[1]
    nh = jnp.maximum(nh_ref[...] @ wu1_ref[...] + agg @ wu2_ref[...]
                     + bu_ref[...], 0.0)
    nh_out_ref[...] = nh
    sec = nh @ w2_ref[...]
    if with_bias:
        sec = sec + b2_ref[...]
    sec_ref[...] = sec


def _node_update(nh, aggp, wu1, wu2, bu, w2, b2, *, with_bias, blk=1024):
    g = N // blk
    k = w2.shape[1]
    return pl.pallas_call(
        functools.partial(_node_update_body, with_bias=with_bias),
        grid=(g,),
        in_specs=[
            pl.BlockSpec((blk, D), lambda i: (i, 0)),
            pl.BlockSpec((2, blk, D), lambda i: (0, i, 0)),
            pl.BlockSpec((D, D), lambda i: (0, 0)),
            pl.BlockSpec((D, D), lambda i: (0, 0)),
            pl.BlockSpec((1, D), lambda i: (0, 0)),
            pl.BlockSpec((D, k), lambda i: (0, 0)),
            pl.BlockSpec((1, k), lambda i: (0, 0)),
        ],
        out_specs=[
            pl.BlockSpec((blk, D), lambda i: (i, 0)),
            pl.BlockSpec((blk, k), lambda i: (i, 0)),
        ],
        out_shape=[
            jax.ShapeDtypeStruct((N, D), jnp.float32),
            jax.ShapeDtypeStruct((N, k), jnp.float32),
        ],
    )(nh, aggp, wu1, wu2, bu.reshape(1, D), w2, b2.reshape(1, k))


def _node_update_last_body(nh_ref, agg_ref, wu1_ref, wu2_ref, bu_ref,
                           wih_ref, bih_ref, gr_ref, gz_ref, gn_ref):
    agg = agg_ref[0] + agg_ref[1]
    nh = jnp.maximum(nh_ref[...] @ wu1_ref[...] + agg @ wu2_ref[...]
                     + bu_ref[...], 0.0)
    wih = wih_ref[...]
    bih = bih_ref[...]
    gr_ref[...] = nh @ wih[:, :H] + bih[:, :H]
    gz_ref[...] = nh @ wih[:, H:2 * H] + bih[:, H:2 * H]
    gn_ref[...] = nh @ wih[:, 2 * H:] + bih[:, 2 * H:]


def _node_update_last(nh, aggp, wu1, wu2, bu, wih, bih, blk=1024):
    g = N // blk
    return pl.pallas_call(
        _node_update_last_body,
        grid=(g,),
        in_specs=[
            pl.BlockSpec((blk, D), lambda i: (i, 0)),
            pl.BlockSpec((2, blk, D), lambda i: (0, i, 0)),
            pl.BlockSpec((D, D), lambda i: (0, 0)),
            pl.BlockSpec((D, D), lambda i: (0, 0)),
            pl.BlockSpec((1, D), lambda i: (0, 0)),
            pl.BlockSpec((D, 3 * H), lambda i: (0, 0)),
            pl.BlockSpec((1, 3 * H), lambda i: (0, 0)),
        ],
        out_specs=[pl.BlockSpec((blk, H), lambda i: (i, 0))] * 3,
        out_shape=[jax.ShapeDtypeStruct((N, H), jnp.float32)] * 3,
    )(nh, aggp, wu1, wu2, bu.reshape(1, D), wih, bih.reshape(1, 3 * H))


def _mol_proj_body(mol_ref, w_ref, b_ref, out_ref):
    out_ref[...] = mol_ref[...] @ w_ref[...] + b_ref[...]


def _mol_proj(mol, w1a, b1):
    return pl.pallas_call(
        _mol_proj_body,
        in_specs=[
            pl.BlockSpec((B, MA), lambda: (0, 0)),
            pl.BlockSpec((MA, MH), lambda: (0, 0)),
            pl.BlockSpec((1, MH), lambda: (0, 0)),
        ],
        out_specs=pl.BlockSpec((B, MH), lambda: (0, 0)),
        out_shape=jax.ShapeDtypeStruct((B, MH), jnp.float32),
    )(mol, w1a, b1.reshape(1, MH))


NC, NS, LANES = 2, 16, 16
NW = NC * NS
_SC_MESH = dict(core_axis_name="c", subcore_axis_name="s")


def _worker_id():
    return lax.axis_index("c") * NS + lax.axis_index("s")


_E_PER_W = E // NW
_ECHUNK = 64
_N_ECHUNKS = _E_PER_W // _ECHUNK


def _mpn_edge_body(p_hbm, ehp_hbm, src_hbm, dst_hbm, aggp_hbm,
                   idxs_v, idxd_v, rows_v, ehp_v, zrow_v, agg_sh, sem):
    c = lax.axis_index("c")
    s = lax.axis_index("s")
    w = c * NS + s
    nrows = N // NS

    def zero_body(j, _):
        zrow_v[j // (D // LANES), pl.ds((j % (D // LANES)) * LANES, LANES)] = (
            jnp.zeros((LANES,), jnp.float32))
        return 0
    lax.fori_loop(0, _ECHUNK * D // LANES, zero_body, 0)
    for k in range(nrows // _ECHUNK):
        pltpu.sync_copy(zrow_v, agg_sh.at[pl.ds(s * nrows + k * _ECHUNK,
                                                _ECHUNK)])
    plsc.subcore_barrier()

    def chunk_body(k, _):
        base = w * _E_PER_W + k * _ECHUNK
        pltpu.sync_copy(src_hbm.at[pl.ds(base, _ECHUNK)], idxs_v)
        pltpu.async_copy(p_hbm.at[idxs_v], rows_v, sem).wait()
        pltpu.sync_copy(ehp_hbm.at[pl.ds(base, _ECHUNK)], ehp_v)

        def ew(j, _):
            r = j // (D // LANES)
            l = (j % (D // LANES)) * LANES
            rows_v[r, pl.ds(l, LANES)] = jnp.maximum(
                rows_v[r, pl.ds(l, LANES)] + ehp_v[r, pl.ds(l, LANES)], 0.0)
            return 0
        lax.fori_loop(0, _ECHUNK * D // LANES, ew, 0)

        pltpu.sync_copy(dst_hbm.at[pl.ds(base, _ECHUNK)], idxd_v)
        pltpu.sync_copy(rows_v, agg_sh.at[idxd_v], add=True)
        return 0
    lax.fori_loop(0, _N_ECHUNKS, chunk_body, 0)
    plsc.subcore_barrier()

    for k in range(nrows // _ECHUNK):
        r0 = s * nrows + k * _ECHUNK
        pltpu.sync_copy(agg_sh.at[pl.ds(r0, _ECHUNK)],
                        aggp_hbm.at[c, pl.ds(r0, _ECHUNK)])


def _mpn_edge_pass(p, ehp, src, dst):
    f = pl.kernel(
        _mpn_edge_body,
        out_type=jax.ShapeDtypeStruct((NC, N, D), jnp.float32),
        mesh=plsc.VectorSubcoreMesh(**_SC_MESH),
        compiler_params=pltpu.CompilerParams(use_tc_tiling_on_sc=False, needs_layout_passes=False),
        scratch_types=[
            pltpu.VMEM((_ECHUNK,), jnp.int32),
            pltpu.VMEM((_ECHUNK,), jnp.int32),
            pltpu.VMEM((_ECHUNK, D), jnp.float32),
            pltpu.VMEM((_ECHUNK, D), jnp.float32),
            pltpu.VMEM((_ECHUNK, D), jnp.float32),
            pltpu.VMEM_SHARED((N, D), jnp.float32),
            pltpu.SemaphoreType.DMA,
        ],
    )
    return f(p, ehp, src, dst)


_N_PER_W = N // NW
_RCHUNK = 80
_N_RCHUNKS = _N_PER_W // _RCHUNK
_B_PER_W = B // NW


def _reorder_body(gr_hbm, gz_hbm, gn_hbm, iof_hbm, gid_hbm, invpi_hbm,
                  sot_hbm, xr_hbm, xz_hbm, xn_hbm, pos_hbm,
                  sot_v, inv_v, iof_v, gid_v, pos_v, rr_v, rz_v, rn_v, sem):
    w = _worker_id()
    pltpu.sync_copy(sot_hbm, sot_v)
    pltpu.sync_copy(invpi_hbm, inv_v)

    def chunk_body(k, _):
        base = w * _N_PER_W + k * _RCHUNK
        pltpu.sync_copy(iof_hbm.at[pl.ds(base, _RCHUNK)], iof_v)
        pltpu.sync_copy(gid_hbm.at[pl.ds(base, _RCHUNK)], gid_v)
        for j in range(_RCHUNK // LANES):
            sl = pl.ds(j * LANES, LANES)
            so16 = plsc.load_gather(sot_v, [iof_v[sl]])
            iv16 = plsc.load_gather(inv_v, [gid_v[sl]])
            pos_v[sl] = so16 + iv16
        pltpu.sync_copy(pos_v, pos_hbm.at[pl.ds(base, _RCHUNK)])
        pltpu.sync_copy(gr_hbm.at[pl.ds(base, _RCHUNK)], rr_v)
        pltpu.sync_copy(gz_hbm.at[pl.ds(base, _RCHUNK)], rz_v)
        pltpu.sync_copy(gn_hbm.at[pl.ds(base, _RCHUNK)], rn_v)
        cps = [pltpu.async_copy(rr_v, xr_hbm.at[pos_v], sem),
               pltpu.async_copy(rz_v, xz_hbm.at[pos_v], sem),
               pltpu.async_copy(rn_v, xn_hbm.at[pos_v], sem)]
        for cp in cps:
            cp.wait()
        return 0
    lax.fori_loop(0, _N_RCHUNKS, chunk_body, 0)


def _reorder(gr, gz, gn, i_of, gid, invpi, sotab):
    f = pl.kernel(
        _reorder_body,
        out_type=[
            jax.ShapeDtypeStruct((NPAD, H), jnp.float32),
            jax.ShapeDtypeStruct((NPAD, H), jnp.float32),
            jax.ShapeDtypeStruct((NPAD, H), jnp.float32),
            jax.ShapeDtypeStruct((N,), jnp.int32),
        ],
        mesh=plsc.VectorSubcoreMesh(**_SC_MESH),
        compiler_params=pltpu.CompilerParams(use_tc_tiling_on_sc=False, needs_layout_passes=False),
        scratch_types=[
            pltpu.VMEM((NT,), jnp.int32),
            pltpu.VMEM((B,), jnp.int32),
            pltpu.VMEM((_RCHUNK,), jnp.int32),
            pltpu.VMEM((_RCHUNK,), jnp.int32),
            pltpu.VMEM((_RCHUNK,), jnp.int32),
            pltpu.VMEM((_RCHUNK, H), jnp.float32),
            pltpu.VMEM((_RCHUNK, H), jnp.float32),
            pltpu.VMEM((_RCHUNK, H), jnp.float32),
            pltpu.SemaphoreType.DMA,
        ],
    )
    return f(gr, gz, gn, i_of, gid, invpi, sotab)


def _final_gather_body(hc_hbm, pos_hbm, molp_hbm, gid_hbm, hn_hbm, mn_hbm,
                       idx_v, hbuf_v, mbuf_v, sem):
    w = _worker_id()

    def chunk_body(k, _):
        base = w * _N_PER_W + k * _RCHUNK
        pltpu.sync_copy(pos_hbm.at[pl.ds(base, _RCHUNK)], idx_v)
        pltpu.async_copy(hc_hbm.at[idx_v], hbuf_v, sem).wait()
        pltpu.sync_copy(hbuf_v, hn_hbm.at[pl.ds(base, _RCHUNK)])
        pltpu.sync_copy(gid_hbm.at[pl.ds(base, _RCHUNK)], idx_v)
        pltpu.async_copy(molp_hbm.at[idx_v], mbuf_v, sem).wait()
        pltpu.sync_copy(mbuf_v, mn_hbm.at[pl.ds(base, _RCHUNK)])
        return 0
    lax.fori_loop(0, _N_RCHUNKS, chunk_body, 0)


def _final_gather(hc, pos, molp, gid):
    f = pl.kernel(
        _final_gather_body,
        out_type=[
            jax.ShapeDtypeStruct((N, H), jnp.float32),
            jax.ShapeDtypeStruct((N, MH), jnp.float32),
        ],
        mesh=plsc.VectorSubcoreMesh(**_SC_MESH),
        compiler_params=pltpu.CompilerParams(use_tc_tiling_on_sc=False, needs_layout_passes=False),
        scratch_types=[
            pltpu.VMEM((_RCHUNK,), jnp.int32),
            pltpu.VMEM((_RCHUNK, H), jnp.float32),
            pltpu.VMEM((_RCHUNK, MH), jnp.float32),
            pltpu.SemaphoreType.DMA,
        ],
    )
    return f(hc, pos, molp, gid)


def _final_mlp_body(hn_ref, mn_ref, w1h_ref, w2_ref, b2_ref, out_ref):
    hid = jnp.maximum(mn_ref[...] + hn_ref[...] @ w1h_ref[...], 0.0)
    out_ref[...] = jax.nn.sigmoid(hid @ w2_ref[...] + b2_ref[...])


def _final_mlp(hn, mn, w1h, w2, b2, blk=1600):
    g = N // blk
    return pl.pallas_call(
        _final_mlp_body,
        grid=(g,),
        in_specs=[
            pl.BlockSpec((blk, H), lambda i: (i, 0)),
            pl.BlockSpec((blk, MH), lambda i: (i, 0)),
            pl.BlockSpec((H, MH), lambda i: (0, 0)),
            pl.BlockSpec((MH, 1), lambda i: (0, 0)),
            pl.BlockSpec((1, 1), lambda i: (0, 0)),
        ],
        out_specs=pl.BlockSpec((blk, 1), lambda i: (i, 0)),
        out_shape=jax.ShapeDtypeStruct((N, 1), jnp.float32),
    )(hn, mn, w1h, w2, b2.reshape(1, 1))


def _gru_body(so_ref, maxi_ref, xr_hbm, xz_hbm, xn_hbm, cnt_ref,
              wr_ref, wz_ref, wn_ref, br_ref, bz_ref, bn_ref, hc_ref,
              h_ref, b0r, b0z, b0n, b1r, b1z, b1n, semA, semB, sem_o):
    maxi = maxi_ref[0]
    h_ref[...] = jnp.zeros((B, H), jnp.float32)
    cnt = cnt_ref[...]
    wr = wr_ref[...]
    wz = wz_ref[...]
    wn = wn_ref[...]
    br = br_ref[...]
    bz = bz_ref[...]
    bn = bn_ref[...]
    bufs = ((b0r, b0z, b0n), (b1r, b1z, b1n))
    sems = (semA, semB)

    def dma3(i, p):
        off = pl.multiple_of(so_ref[i], 8)
        return [pltpu.make_async_copy(srcs.at[pl.ds(off, B)], buf, sems[p])
                for srcs, buf in zip((xr_hbm, xz_hbm, xn_hbm), bufs[p])]

    def start3(i, p):
        for cp in dma3(i, p):
            cp.start()

    def wait3(i, p):
        for cp in dma3(i, p):
            cp.wait()

    def half(i, p, last):
        @pl.when(i < maxi)
        def _():
            wait3(i, p)
            off = pl.multiple_of(so_ref[i], 8)
            h = h_ref[...]
            gr = h @ wr + br + bufs[p][0][...]
            gz = h @ wz + bz + bufs[p][1][...]
            ghn = h @ wn + bn
            r = 0.5 * jnp.tanh(0.5 * gr) + 0.5
            z = 0.5 * jnp.tanh(0.5 * gz) + 0.5
            n = jnp.tanh(bufs[p][2][...] + r * ghn)
            mask = i < cnt
            h_ref[...] = jnp.where(mask, (1.0 - z) * n + z * h, h)
            if last:
                cp = pltpu.make_async_copy(h_ref, hc_ref.at[pl.ds(off, B)],
                                           sem_o)
                cp.start()
                cp.wait()
            start3(i + 2, p)

    for t in range(N_RNN_ITERS + 1):
        start3(0, 0)
        start3(1, 1)

        def pair(k, _, last=(t == N_RNN_ITERS)):
            half(2 * k, 0, last)
            half(2 * k + 1, 1, last)
            return 0

        lax.fori_loop(0, (maxi + 1) // 2, pair, 0)
        wait3(0, 0)
        wait3(1, 1)


def _gru_phase(sotab, maxi, xr, xz, xn, counts_sorted, whh, bhh):
    return pl.pallas_call(
        _gru_body,
        in_specs=[
            pl.BlockSpec(memory_space=pltpu.SMEM),
            pl.BlockSpec(memory_space=pltpu.SMEM),
            pl.BlockSpec(memory_space=pl.ANY),
            pl.BlockSpec(memory_space=pl.ANY),
            pl.BlockSpec(memory_space=pl.ANY),
            pl.BlockSpec((B, 1), lambda: (0, 0)),
            pl.BlockSpec((H, H), lambda: (0, 0)),
            pl.BlockSpec((H, H), lambda: (0, 0)),
            pl.BlockSpec((H, H), lambda: (0, 0)),
            pl.BlockSpec((1, H), lambda: (0, 0)),
            pl.BlockSpec((1, H), lambda: (0, 0)),
            pl.BlockSpec((1, H), lambda: (0, 0)),
        ],
        out_specs=pl.BlockSpec(memory_space=pl.ANY),
        out_shape=jax.ShapeDtypeStruct((NPAD, H), jnp.float32),
        scratch_shapes=[
            pltpu.VMEM((B, H), jnp.float32),
            pltpu.VMEM((B, H), jnp.float32),
            pltpu.VMEM((B, H), jnp.float32),
            pltpu.VMEM((B, H), jnp.float32),
            pltpu.VMEM((B, H), jnp.float32),
            pltpu.VMEM((B, H), jnp.float32),
            pltpu.VMEM((B, H), jnp.float32),
            pltpu.SemaphoreType.DMA,
            pltpu.SemaphoreType.DMA,
            pltpu.SemaphoreType.DMA,
        ],
    )(sotab, maxi.reshape(1), xr, xz, xn, counts_sorted.reshape(B, 1),
      whh[:, :H], whh[:, H:2 * H], whh[:, 2 * H:],
      bhh[:H].reshape(1, H), bhh[H:2 * H].reshape(1, H),
      bhh[2 * H:].reshape(1, H))


def kernel(mol_a_reprs, node_feats, edge_feats, edge_index, node_graph_ids,
           W_node, b_node, W_edge, b_edge, W_msg, b_msg, W_upd, b_upd,
           W_ih, W_hh, b_ih, b_hh, W1, b1, W2, b2):
    src = edge_index[0]
    dst = edge_index[1]
    gid = node_graph_ids

    iota_n = jnp.arange(N, dtype=jnp.int32)
    counts = jnp.sum(
        (gid[None, :] == jnp.arange(B, dtype=gid.dtype)[:, None]
         ).astype(jnp.int32), axis=1, dtype=jnp.int32)
    maxi = jnp.max(counts).astype(jnp.int32)
    pi = jnp.argsort(-counts).astype(jnp.int32)
    counts_sorted = -jnp.sort(-counts)
    invpi = jnp.argsort(pi).astype(jnp.int32)
    a = jnp.sum(
        (counts[None, :] > jnp.arange(NT, dtype=jnp.int32)[:, None]
         ).astype(jnp.int32), axis=1, dtype=jnp.int32)
    a_al = ((a + 7) // 8) * 8
    sotab = (jnp.cumsum(a_al) - a_al).astype(jnp.int32)
    bm = jnp.concatenate([jnp.ones((1,), jnp.bool_), gid[1:] != gid[:-1]])
    seg_start = jax.lax.cummax(jnp.where(bm, iota_n, 0))
    i_of = iota_n - seg_start

    wmt = W_msg[:D]
    wmb = W_msg[D:]
    wu1 = W_upd[:D]
    wu2 = W_upd[D:]
    w1a = W1[:MA]
    w1h = W1[MA:]

    nh, p = _node_embed(node_feats, W_node, b_node, wmt)
    ehp = _edge_embed(edge_feats, W_edge, b_edge, wmb, b_msg)
    for s in range(N_MPN_STEPS):
        aggp = _mpn_edge_pass(p, ehp, src, dst)
        if s == N_MPN_STEPS - 1:
            gr, gz, gn = _node_update_last(nh, aggp, wu1, wu2, b_upd,
                                           W_ih, b_ih)
        else:
            nh, p = _node_update(nh, aggp, wu1, wu2, b_upd, wmt,
                                 jnp.zeros((D,), jnp.float32),
                                 with_bias=False)

    molp = _mol_proj(mol_a_reprs, w1a, b1)
    xr, xz, xn, pos = _reorder(gr, gz, gn, i_of, gid, invpi, sotab)
    hc = _gru_phase(sotab, maxi, xr, xz, xn, counts_sorted, W_hh, b_hh)
    hn, mn = _final_gather(hc, pos, molp, gid)
    return _final_mlp(hn, mn, w1h, W2, b2).reshape(N)


# speedup vs baseline: 2.9755x; 1.2797x over previous
import functools
import jax
import jax.numpy as jnp
from jax import lax
from jax.experimental import pallas as pl
from jax.experimental.pallas import tpu as pltpu
from jax.experimental.pallas import tpu_sc as plsc

N_MPN_STEPS = 4
N_RNN_ITERS = 2

B, MA = 1024, 128
N, E = 25600, 51200
NF, EF = 64, 16
D, EH = 64, 32
H, MH = 64, 128
NPAD = 8 * N + B
NT = N + 8


def _node_embed_body(nf_ref, wn_ref, bn_ref, wmt_ref, nh_ref, p_ref):
    nh = jnp.maximum(nf_ref[...] @ wn_ref[...] + bn_ref[...], 0.0)
    nh_ref[...] = nh
    p_ref[...] = nh @ wmt_ref[...]


def _node_embed(nf, wn, bn, wmt, blk=1024):
    g = N // blk
    return pl.pallas_call(
        _node_embed_body,
        grid=(g,),
        in_specs=[
            pl.BlockSpec((blk, NF), lambda i: (i, 0)),
            pl.BlockSpec((NF, D), lambda i: (0, 0)),
            pl.BlockSpec((1, D), lambda i: (0, 0)),
            pl.BlockSpec((D, D), lambda i: (0, 0)),
        ],
        out_specs=[
            pl.BlockSpec((blk, D), lambda i: (i, 0)),
            pl.BlockSpec((blk, D), lambda i: (i, 0)),
        ],
        out_shape=[
            jax.ShapeDtypeStruct((N, D), jnp.float32),
            jax.ShapeDtypeStruct((N, D), jnp.float32),
        ],
    )(nf, wn, bn.reshape(1, D), wmt)


def _edge_embed_body(ef_ref, we_ref, be_ref, wmb_ref, bm_ref, ehp_ref):
    eh = jnp.maximum(ef_ref[...] @ we_ref[...] + be_ref[...], 0.0)
    ehp_ref[...] = eh @ wmb_ref[...] + bm_ref[...]


def _edge_embed(ef, we, be, wmb, bm, blk=2048):
    g = E // blk
    return pl.pallas_call(
        _edge_embed_body,
        grid=(g,),
        in_specs=[
            pl.BlockSpec((blk, EF), lambda i: (i, 0)),
            pl.BlockSpec((EF, EH), lambda i: (0, 0)),
            pl.BlockSpec((1, EH), lambda i: (0, 0)),
            pl.BlockSpec((EH, D), lambda i: (0, 0)),
            pl.BlockSpec((1, D), lambda i: (0, 0)),
        ],
        out_specs=pl.BlockSpec((blk, D), lambda i: (i, 0)),
        out_shape=jax.ShapeDtypeStruct((E, D), jnp.float32),
    )(ef, we, be.reshape(1, EH), wmb, bm.reshape(1, D))


def _node_update_body(nh_ref, agg_ref, wu1_ref, wu2_ref, bu_ref, w2_ref,
                      b2_ref, nh_out_ref, sec_ref, *, with_bias):
    agg = agg_ref[0] + agg_ref[1]
    nh = jnp.maximum(nh_ref[...] @ wu1_ref[...] + agg @ wu2_ref[...]
                     + bu_ref[...], 0.0)
    nh_out_ref[...] = nh
    sec = nh @ w2_ref[...]
    if with_bias:
        sec = sec + b2_ref[...]
    sec_ref[...] = sec


def _node_update(nh, aggp, wu1, wu2, bu, w2, b2, *, with_bias, blk=1024):
    g = N // blk
    k = w2.shape[1]
    return pl.pallas_call(
        functools.partial(_node_update_body, with_bias=with_bias),
        grid=(g,),
        in_specs=[
            pl.BlockSpec((blk, D), lambda i: (i, 0)),
            pl.BlockSpec((2, blk, D), lambda i: (0, i, 0)),
            pl.BlockSpec((D, D), lambda i: (0, 0)),
            pl.BlockSpec((D, D), lambda i: (0, 0)),
            pl.BlockSpec((1, D), lambda i: (0, 0)),
            pl.BlockSpec((D, k), lambda i: (0, 0)),
            pl.BlockSpec((1, k), lambda i: (0, 0)),
        ],
        out_specs=[
            pl.BlockSpec((blk, D), lambda i: (i, 0)),
            pl.BlockSpec((blk, k), lambda i: (i, 0)),
        ],
        out_shape=[
            jax.ShapeDtypeStruct((N, D), jnp.float32),
            jax.ShapeDtypeStruct((N, k), jnp.float32),
        ],
    )(nh, aggp, wu1, wu2, bu.reshape(1, D), w2, b2.reshape(1, k))


def _node_update_last_body(nh_ref, agg_ref, wu1_ref, wu2_ref, bu_ref,
                           wih_ref, bih_ref, gr_ref, gz_ref, gn_ref):
    agg = agg_ref[0] + agg_ref[1]
    nh = jnp.maximum(nh_ref[...] @ wu1_ref[...] + agg @ wu2_ref[...]
                     + bu_ref[...], 0.0)
    wih = wih_ref[...]
    bih = bih_ref[...]
    gr_ref[...] = nh @ wih[:, :H] + bih[:, :H]
    gz_ref[...] = nh @ wih[:, H:2 * H] + bih[:, H:2 * H]
    gn_ref[...] = nh @ wih[:, 2 * H:] + bih[:, 2 * H:]


def _node_update_last(nh, aggp, wu1, wu2, bu, wih, bih, blk=1024):
    g = N // blk
    return pl.pallas_call(
        _node_update_last_body,
        grid=(g,),
        in_specs=[
            pl.BlockSpec((blk, D), lambda i: (i, 0)),
            pl.BlockSpec((2, blk, D), lambda i: (0, i, 0)),
            pl.BlockSpec((D, D), lambda i: (0, 0)),
            pl.BlockSpec((D, D), lambda i: (0, 0)),
            pl.BlockSpec((1, D), lambda i: (0, 0)),
            pl.BlockSpec((D, 3 * H), lambda i: (0, 0)),
            pl.BlockSpec((1, 3 * H), lambda i: (0, 0)),
        ],
        out_specs=[pl.BlockSpec((blk, H), lambda i: (i, 0))] * 3,
        out_shape=[jax.ShapeDtypeStruct((N, H), jnp.float32)] * 3,
    )(nh, aggp, wu1, wu2, bu.reshape(1, D), wih, bih.reshape(1, 3 * H))


def _mol_proj_body(mol_ref, w_ref, b_ref, out_ref):
    out_ref[...] = mol_ref[...] @ w_ref[...] + b_ref[...]


def _mol_proj(mol, w1a, b1):
    return pl.pallas_call(
        _mol_proj_body,
        in_specs=[
            pl.BlockSpec((B, MA), lambda: (0, 0)),
            pl.BlockSpec((MA, MH), lambda: (0, 0)),
            pl.BlockSpec((1, MH), lambda: (0, 0)),
        ],
        out_specs=pl.BlockSpec((B, MH), lambda: (0, 0)),
        out_shape=jax.ShapeDtypeStruct((B, MH), jnp.float32),
    )(mol, w1a, b1.reshape(1, MH))


NC, NS, LANES = 2, 16, 16
NW = NC * NS
_SC_MESH = dict(core_axis_name="c", subcore_axis_name="s")


def _worker_id():
    return lax.axis_index("c") * NS + lax.axis_index("s")


_E_PER_W = E // NW
_ECHUNK = 40
_N_ECHUNKS = _E_PER_W // _ECHUNK
_NBUF = 3


def _mpn_edge_body(p_hbm, ehp_hbm, src_hbm, dst_hbm, aggp_hbm,
                   idxs_v, idxd_v, rows0_v, rows1_v, rows2_v,
                   ehp0_v, ehp1_v, ehp2_v, agg_sh,
                   seml0, seml1, seml2, semsc0, semsc1, semsc2):
    c = lax.axis_index("c")
    s = lax.axis_index("s")
    w = c * NS + s
    ebase = w * _E_PER_W
    nrows = N // NS
    rows = (rows0_v, rows1_v, rows2_v)
    ehps = (ehp0_v, ehp1_v, ehp2_v)
    seml = (seml0, seml1, seml2)
    semsc = (semsc0, semsc1, semsc2)

    pltpu.sync_copy(src_hbm.at[pl.ds(ebase, _E_PER_W)], idxs_v)

    def zero_body(j, _):
        rows0_v[j // (D // LANES), pl.ds((j % (D // LANES)) * LANES, LANES)] = (
            jnp.zeros((LANES,), jnp.float32))
        return 0
    lax.fori_loop(0, _ECHUNK * D // LANES, zero_body, 0)
    for k in range(nrows // _ECHUNK):
        pltpu.sync_copy(rows0_v, agg_sh.at[pl.ds(s * nrows + k * _ECHUNK,
                                                 _ECHUNK)])
    plsc.subcore_barrier()

    pend_ld = {}
    pend_sc = {}

    def issue(k):
        par = k % _NBUF
        base = ebase + k * _ECHUNK
        pend_ld[k] = (
            pltpu.async_copy(p_hbm.at[idxs_v.at[pl.ds(k * _ECHUNK, _ECHUNK)]],
                             rows[par], seml[par]),
            pltpu.async_copy(ehp_hbm.at[pl.ds(base, _ECHUNK)], ehps[par],
                             seml[par]),
            pltpu.async_copy(dst_hbm.at[pl.ds(base, _ECHUNK)],
                             idxd_v.at[par], seml[par]))

    def consume(k):
        par = k % _NBUF
        for cp in pend_ld.pop(k):
            cp.wait()

        def ew(r, _):
            for l in range(D // LANES):
                sl = pl.ds(l * LANES, LANES)
                rows[par][r, sl] = jnp.maximum(
                    rows[par][r, sl] + ehps[par][r, sl], 0.0)
            return 0
        lax.fori_loop(0, _ECHUNK, ew, 0)
        pend_sc[k] = pltpu.async_copy(rows[par], agg_sh.at[idxd_v.at[par]],
                                      semsc[par], add=True)

    issue(0)
    issue(1)
    for k in range(_N_ECHUNKS):
        consume(k)
        if k + 2 < _N_ECHUNKS:
            if k >= 1:
                pend_sc.pop(k - 1).wait()
            issue(k + 2)
    for k in sorted(pend_sc):
        pend_sc[k].wait()
    pend_sc.clear()
    plsc.subcore_barrier()

    for k in range(nrows // _ECHUNK):
        r0 = s * nrows + k * _ECHUNK
        pltpu.sync_copy(agg_sh.at[pl.ds(r0, _ECHUNK)],
                        aggp_hbm.at[c, pl.ds(r0, _ECHUNK)])


def _mpn_edge_pass(p, ehp, src, dst):
    f = pl.kernel(
        _mpn_edge_body,
        out_type=jax.ShapeDtypeStruct((NC, N, D), jnp.float32),
        mesh=plsc.VectorSubcoreMesh(**_SC_MESH),
        compiler_params=pltpu.CompilerParams(use_tc_tiling_on_sc=False, needs_layout_passes=False),
        scratch_types=[
            pltpu.VMEM((_E_PER_W,), jnp.int32),
            pltpu.VMEM((_NBUF, _ECHUNK), jnp.int32),
            pltpu.VMEM((_ECHUNK, D), jnp.float32),
            pltpu.VMEM((_ECHUNK, D), jnp.float32),
            pltpu.VMEM((_ECHUNK, D), jnp.float32),
            pltpu.VMEM((_ECHUNK, D), jnp.float32),
            pltpu.VMEM((_ECHUNK, D), jnp.float32),
            pltpu.VMEM((_ECHUNK, D), jnp.float32),
            pltpu.VMEM_SHARED((N, D), jnp.float32),
            pltpu.SemaphoreType.DMA,
            pltpu.SemaphoreType.DMA,
            pltpu.SemaphoreType.DMA,
            pltpu.SemaphoreType.DMA,
            pltpu.SemaphoreType.DMA,
            pltpu.SemaphoreType.DMA,
        ],
    )
    return f(p, ehp, src, dst)


_N_PER_W = N // NW
_RCHUNK = 80
_N_RCHUNKS = _N_PER_W // _RCHUNK
_B_PER_W = B // NW


def _reorder_body(gr_hbm, gz_hbm, gn_hbm, iof_hbm, gid_hbm, invpi_hbm,
                  sot_hbm, xr_hbm, xz_hbm, xn_hbm, pos_hbm,
                  sot_v, inv_v, iof_v, gid_v, pos2_v,
                  r0_v, r1_v, r2_v, z0_v, z1_v, z2_v, n0_v, n1_v, n2_v,
                  seml0, seml1, seml2, semsc0, semsc1, semsc2, semp):
    w = _worker_id()
    base = w * _N_PER_W
    rb = (r0_v, r1_v, r2_v)
    zb = (z0_v, z1_v, z2_v)
    nb = (n0_v, n1_v, n2_v)
    seml = (seml0, seml1, seml2)
    semsc = (semsc0, semsc1, semsc2)

    pltpu.sync_copy(sot_hbm, sot_v)
    pltpu.sync_copy(invpi_hbm, inv_v)
    pltpu.sync_copy(iof_hbm.at[pl.ds(base, _N_PER_W)], iof_v)
    pltpu.sync_copy(gid_hbm.at[pl.ds(base, _N_PER_W)], gid_v)

    for k in range(_N_RCHUNKS):
        for j in range(_RCHUNK // LANES):
            sl = pl.ds(k * _RCHUNK + j * LANES, LANES)
            so16 = plsc.load_gather(sot_v, [iof_v[sl]])
            iv16 = plsc.load_gather(inv_v, [gid_v[sl]])
            pos2_v[k, pl.ds(j * LANES, LANES)] = so16 + iv16
    pend_p = [pltpu.async_copy(pos2_v.at[k],
                               pos_hbm.at[pl.ds(base + k * _RCHUNK, _RCHUNK)],
                               semp)
              for k in range(_N_RCHUNKS)]

    pend_ld = {}
    pend_sc = {}

    def issue(k):
        par = k % 3
        sl = pl.ds(base + k * _RCHUNK, _RCHUNK)
        pend_ld[k] = (
            pltpu.async_copy(gr_hbm.at[sl], rb[par], seml[par]),
            pltpu.async_copy(gz_hbm.at[sl], zb[par], seml[par]),
            pltpu.async_copy(gn_hbm.at[sl], nb[par], seml[par]))

    def consume(k):
        par = k % 3
        for cp in pend_ld.pop(k):
            cp.wait()
        idx = pos2_v.at[k]
        pend_sc[k] = (
            pltpu.async_copy(rb[par], xr_hbm.at[idx], semsc[par]),
            pltpu.async_copy(zb[par], xz_hbm.at[idx], semsc[par]),
            pltpu.async_copy(nb[par], xn_hbm.at[idx], semsc[par]))

    issue(0)
    issue(1)
    for k in range(_N_RCHUNKS):
        consume(k)
        if k + 2 < _N_RCHUNKS:
            if k >= 1:
                for cp in pend_sc.pop(k - 1):
                    cp.wait()
            issue(k + 2)
    for k in sorted(pend_sc):
        for cp in pend_sc[k]:
            cp.wait()
    for cp in pend_p:
        cp.wait()


def _reorder(gr, gz, gn, i_of, gid, invpi, sotab):
    f = pl.kernel(
        _reorder_body,
        out_type=[
            jax.ShapeDtypeStruct((NPAD, H), jnp.float32),
            jax.ShapeDtypeStruct((NPAD, H), jnp.float32),
            jax.ShapeDtypeStruct((NPAD, H), jnp.float32),
            jax.ShapeDtypeStruct((N,), jnp.int32),
        ],
        mesh=plsc.VectorSubcoreMesh(**_SC_MESH),
        compiler_params=pltpu.CompilerParams(use_tc_tiling_on_sc=False, needs_layout_passes=False),
        scratch_types=[
            pltpu.VMEM((NT,), jnp.int32),
            pltpu.VMEM((B,), jnp.int32),
            pltpu.VMEM((_N_PER_W,), jnp.int32),
            pltpu.VMEM((_N_PER_W,), jnp.int32),
            pltpu.VMEM((_N_RCHUNKS, _RCHUNK), jnp.int32),
            pltpu.VMEM((_RCHUNK, H), jnp.float32),
            pltpu.VMEM((_RCHUNK, H), jnp.float32),
            pltpu.VMEM((_RCHUNK, H), jnp.float32),
            pltpu.VMEM((_RCHUNK, H), jnp.float32),
            pltpu.VMEM((_RCHUNK, H), jnp.float32),
            pltpu.VMEM((_RCHUNK, H), jnp.float32),
            pltpu.VMEM((_RCHUNK, H), jnp.float32),
            pltpu.VMEM((_RCHUNK, H), jnp.float32),
            pltpu.VMEM((_RCHUNK, H), jnp.float32),
            pltpu.SemaphoreType.DMA,
            pltpu.SemaphoreType.DMA,
            pltpu.SemaphoreType.DMA,
            pltpu.SemaphoreType.DMA,
            pltpu.SemaphoreType.DMA,
            pltpu.SemaphoreType.DMA,
            pltpu.SemaphoreType.DMA,
        ],
    )
    return f(gr, gz, gn, i_of, gid, invpi, sotab)


def _final_gather_body(hc_hbm, pos_hbm, molp_hbm, gid_hbm, hn_hbm, mn_hbm,
                       pos_v, gid_v, h0_v, h1_v, h2_v, m0_v, m1_v, m2_v,
                       semg0, semg1, semg2, semst):
    w = _worker_id()
    base = w * _N_PER_W
    hb = (h0_v, h1_v, h2_v)
    mb = (m0_v, m1_v, m2_v)
    semg = (semg0, semg1, semg2)
    pltpu.sync_copy(pos_hbm.at[pl.ds(base, _N_PER_W)], pos_v)
    pltpu.sync_copy(gid_hbm.at[pl.ds(base, _N_PER_W)], gid_v)

    pend_g = {}
    pend_s = {}

    def issue(k):
        par = k % 3
        sl = pl.ds(k * _RCHUNK, _RCHUNK)
        pend_g[k] = (
            pltpu.async_copy(hc_hbm.at[pos_v.at[sl]], hb[par], semg[par]),
            pltpu.async_copy(molp_hbm.at[gid_v.at[sl]], mb[par], semg[par]))

    def consume(k):
        par = k % 3
        for cp in pend_g.pop(k):
            cp.wait()
        out_sl = pl.ds(base + k * _RCHUNK, _RCHUNK)
        pend_s[k] = (
            pltpu.async_copy(hb[par], hn_hbm.at[out_sl], semst),
            pltpu.async_copy(mb[par], mn_hbm.at[out_sl], semst))

    issue(0)
    issue(1)
    for k in range(_N_RCHUNKS):
        consume(k)
        if k + 2 < _N_RCHUNKS:
            if k >= 1:
                for cp in pend_s.pop(k - 1):
                    cp.wait()
            issue(k + 2)
    for k in sorted(pend_s):
        for cp in pend_s[k]:
            cp.wait()


def _final_gather(hc, pos, molp, gid):
    f = pl.kernel(
        _final_gather_body,
        out_type=[
            jax.ShapeDtypeStruct((N, H), jnp.float32),
            jax.ShapeDtypeStruct((N, MH), jnp.float32),
        ],
        mesh=plsc.VectorSubcoreMesh(**_SC_MESH),
        compiler_params=pltpu.CompilerParams(use_tc_tiling_on_sc=False, needs_layout_passes=False),
        scratch_types=[
            pltpu.VMEM((_N_PER_W,), jnp.int32),
            pltpu.VMEM((_N_PER_W,), jnp.int32),
            pltpu.VMEM((_RCHUNK, H), jnp.float32),
            pltpu.VMEM((_RCHUNK, H), jnp.float32),
            pltpu.VMEM((_RCHUNK, H), jnp.float32),
            pltpu.VMEM((_RCHUNK, MH), jnp.float32),
            pltpu.VMEM((_RCHUNK, MH), jnp.float32),
            pltpu.VMEM((_RCHUNK, MH), jnp.float32),
            pltpu.SemaphoreType.DMA,
            pltpu.SemaphoreType.DMA,
            pltpu.SemaphoreType.DMA,
            pltpu.SemaphoreType.DMA,
        ],
    )
    return f(hc, pos, molp, gid)


def _final_mlp_body(hn_ref, mn_ref, w1h_ref, w2_ref, b2_ref, out_ref):
    hid = jnp.maximum(mn_ref[...] + hn_ref[...] @ w1h_ref[...], 0.0)
    out_ref[...] = jax.nn.sigmoid(hid @ w2_ref[...] + b2_ref[...])


def _final_mlp(hn, mn, w1h, w2, b2, blk=1600):
    g = N // blk
    return pl.pallas_call(
        _final_mlp_body,
        grid=(g,),
        in_specs=[
            pl.BlockSpec((blk, H), lambda i: (i, 0)),
            pl.BlockSpec((blk, MH), lambda i: (i, 0)),
            pl.BlockSpec((H, MH), lambda i: (0, 0)),
            pl.BlockSpec((MH, 1), lambda i: (0, 0)),
            pl.BlockSpec((1, 1), lambda i: (0, 0)),
        ],
        out_specs=pl.BlockSpec((blk, 1), lambda i: (i, 0)),
        out_shape=jax.ShapeDtypeStruct((N, 1), jnp.float32),
    )(hn, mn, w1h, w2, b2.reshape(1, 1))


_GRU_TIERS = (1024, 512, 256, 128, 64)


def _gru_body(so_ref, maxi_ref, xr_hbm, xz_hbm, xn_hbm, cnt_ref,
              wr_ref, wz_ref, wn_ref, br_ref, bz_ref, bn_ref, hc_ref,
              h_ref, b0r, b0z, b0n, b1r, b1z, b1n, semA, semB, sem_o):
    maxi = maxi_ref[0]
    h_ref[...] = jnp.zeros((B, H), jnp.float32)
    cnt = cnt_ref[...]
    wr = wr_ref[...]
    wz = wz_ref[...]
    wn = wn_ref[...]
    br = br_ref[...]
    bz = bz_ref[...]
    bn = bn_ref[...]
    bufs = ((b0r, b0z, b0n), (b1r, b1z, b1n))
    sems = (semA, semB)

    def for_each_tier(na, fn):
        nt = len(_GRU_TIERS)
        for k, T in enumerate(_GRU_TIERS):
            if k == 0:
                cond = na > _GRU_TIERS[1]
            elif k == nt - 1:
                cond = na <= _GRU_TIERS[k]
            else:
                cond = (na <= _GRU_TIERS[k]) & (na > _GRU_TIERS[k + 1])
            pl.when(cond)(lambda T=T: fn(T))

    def dma3(i, p, T):
        off = pl.multiple_of(so_ref[i], 8)
        return [pltpu.make_async_copy(srcs.at[pl.ds(off, T)],
                                      buf.at[pl.ds(0, T)], sems[p])
                for srcs, buf in zip((xr_hbm, xz_hbm, xn_hbm), bufs[p])]

    def _start_all(i, p, T):
        for cp in dma3(i, p, T):
            cp.start()

    def _wait_all(i, p, T):
        for cp in dma3(i, p, T):
            cp.wait()

    def start3(i, p):
        na = so_ref[i + 1] - so_ref[i]
        for_each_tier(na, lambda T: _start_all(i, p, T))

    def wait3(i, p):
        na = so_ref[i + 1] - so_ref[i]
        for_each_tier(na, lambda T: _wait_all(i, p, T))

    def half(i, p, last):
        @pl.when(i < maxi)
        def _():
            na = so_ref[i + 1] - so_ref[i]
            off = pl.multiple_of(so_ref[i], 8)

            def body(T):
                for cp in dma3(i, p, T):
                    cp.wait()
                sl = pl.ds(0, T)
                h = h_ref[sl]
                gr = h @ wr + br + bufs[p][0][sl]
                gz = h @ wz + bz + bufs[p][1][sl]
                ghn = h @ wn + bn
                r = 0.5 * jnp.tanh(0.5 * gr) + 0.5
                z = 0.5 * jnp.tanh(0.5 * gz) + 0.5
                n = jnp.tanh(bufs[p][2][sl] + r * ghn)
                mask = i < cnt[:T]
                h_ref[sl] = jnp.where(mask, (1.0 - z) * n + z * h, h)
                if last:
                    cp = pltpu.make_async_copy(h_ref.at[sl],
                                               hc_ref.at[pl.ds(off, T)],
                                               sem_o)
                    cp.start()
                    cp.wait()

            for_each_tier(na, body)
            start3(i + 2, p)

    for t in range(N_RNN_ITERS + 1):
        start3(0, 0)
        start3(1, 1)

        def pair(k, _, last=(t == N_RNN_ITERS)):
            half(2 * k, 0, last)
            half(2 * k + 1, 1, last)
            return 0

        lax.fori_loop(0, (maxi + 1) // 2, pair, 0)
        ev = maxi + (maxi % 2)
        od = maxi + 1 - (maxi % 2)
        wait3(ev, 0)
        wait3(od, 1)


def _gru_phase(sotab, maxi, xr, xz, xn, counts_sorted, whh, bhh):
    return pl.pallas_call(
        _gru_body,
        in_specs=[
            pl.BlockSpec(memory_space=pltpu.SMEM),
            pl.BlockSpec(memory_space=pltpu.SMEM),
            pl.BlockSpec(memory_space=pl.ANY),
            pl.BlockSpec(memory_space=pl.ANY),
            pl.BlockSpec(memory_space=pl.ANY),
            pl.BlockSpec((B, 1), lambda: (0, 0)),
            pl.BlockSpec((H, H), lambda: (0, 0)),
            pl.BlockSpec((H, H), lambda: (0, 0)),
            pl.BlockSpec((H, H), lambda: (0, 0)),
            pl.BlockSpec((1, H), lambda: (0, 0)),
            pl.BlockSpec((1, H), lambda: (0, 0)),
            pl.BlockSpec((1, H), lambda: (0, 0)),
        ],
        out_specs=pl.BlockSpec(memory_space=pl.ANY),
        out_shape=jax.ShapeDtypeStruct((NPAD, H), jnp.float32),
        scratch_shapes=[
            pltpu.VMEM((B, H), jnp.float32),
            pltpu.VMEM((B, H), jnp.float32),
            pltpu.VMEM((B, H), jnp.float32),
            pltpu.VMEM((B, H), jnp.float32),
            pltpu.VMEM((B, H), jnp.float32),
            pltpu.VMEM((B, H), jnp.float32),
            pltpu.VMEM((B, H), jnp.float32),
            pltpu.SemaphoreType.DMA,
            pltpu.SemaphoreType.DMA,
            pltpu.SemaphoreType.DMA,
        ],
    )(sotab, maxi.reshape(1), xr, xz, xn, counts_sorted.reshape(B, 1),
      whh[:, :H], whh[:, H:2 * H], whh[:, 2 * H:],
      bhh[:H].reshape(1, H), bhh[H:2 * H].reshape(1, H),
      bhh[2 * H:].reshape(1, H))


def kernel(mol_a_reprs, node_feats, edge_feats, edge_index, node_graph_ids,
           W_node, b_node, W_edge, b_edge, W_msg, b_msg, W_upd, b_upd,
           W_ih, W_hh, b_ih, b_hh, W1, b1, W2, b2):
    src = edge_index[0]
    dst = edge_index[1]
    gid = node_graph_ids

    iota_n = jnp.arange(N, dtype=jnp.int32)
    counts = jnp.sum(
        (gid[None, :] == jnp.arange(B, dtype=gid.dtype)[:, None]
         ).astype(jnp.int32), axis=1, dtype=jnp.int32)
    maxi = jnp.max(counts).astype(jnp.int32)
    pi = jnp.argsort(-counts).astype(jnp.int32)
    counts_sorted = -jnp.sort(-counts)
    invpi = jnp.argsort(pi).astype(jnp.int32)
    a = jnp.sum(
        (counts[None, :] > jnp.arange(NT, dtype=jnp.int32)[:, None]
         ).astype(jnp.int32), axis=1, dtype=jnp.int32)
    a_al = ((a + 7) // 8) * 8
    sotab = (jnp.cumsum(a_al) - a_al).astype(jnp.int32)
    bm = jnp.concatenate([jnp.ones((1,), jnp.bool_), gid[1:] != gid[:-1]])
    seg_start = jax.lax.cummax(jnp.where(bm, iota_n, 0))
    i_of = iota_n - seg_start

    wmt = W_msg[:D]
    wmb = W_msg[D:]
    wu1 = W_upd[:D]
    wu2 = W_upd[D:]
    w1a = W1[:MA]
    w1h = W1[MA:]

    nh, p = _node_embed(node_feats, W_node, b_node, wmt)
    ehp = _edge_embed(edge_feats, W_edge, b_edge, wmb, b_msg)
    for s in range(N_MPN_STEPS):
        aggp = _mpn_edge_pass(p, ehp, src, dst)
        if s == N_MPN_STEPS - 1:
            gr, gz, gn = _node_update_last(nh, aggp, wu1, wu2, b_upd,
                                           W_ih, b_ih)
        else:
            nh, p = _node_update(nh, aggp, wu1, wu2, b_upd, wmt,
                                 jnp.zeros((D,), jnp.float32),
                                 with_bias=False)

    molp = _mol_proj(mol_a_reprs, w1a, b1)
    xr, xz, xn, pos = _reorder(gr, gz, gn, i_of, gid, invpi, sotab)
    hc = _gru_phase(sotab, maxi, xr, xz, xn, counts_sorted, W_hh, b_hh)
    hn, mn = _final_gather(hc, pos, molp, gid)
    return _final_mlp(hn, mn, w1h, W2, b2).reshape(N)
